# baseline (speedup 1.0000x reference)
import jax
import jax.numpy as jnp
from jax import lax
from jax.experimental import pallas as pl
from jax.experimental.pallas import tpu as pltpu
from jax.experimental.pallas import tpu_sc as plsc

_NF, _CARD, _DIM, _B = 26, 100000, 24, 16384
_ROWS = _B * _NF
_NC, _NS = 2, 16
_NW = _NC * _NS
_RPW = _ROWS // _NW
_CHUNK = 128
_K = 8
_GROUPS = _RPW // (_CHUNK * _K)


def _gather_body(idx_hbm, table_hbm, out_hbm, idx_v, rows_v, gsem):
    wid = lax.axis_index("s") * _NC + lax.axis_index("c")
    base = wid * _RPW
    pltpu.sync_copy(idx_hbm.at[wid], idx_v)

    def group(g, carry):
        cps = [
            pltpu.async_copy(
                table_hbm.at[idx_v.at[g * _K + b]],
                rows_v.at[pl.ds(b * _CHUNK, _CHUNK)],
                gsem,
            )
            for b in range(_K)
        ]
        for cp in cps:
            cp.wait()
        pltpu.sync_copy(
            rows_v,
            out_hbm.at[pl.ds(base + g * (_K * _CHUNK), _K * _CHUNK)],
        )
        return carry

    lax.fori_loop(0, _GROUPS, group, 0)


@jax.jit
def _cat_embed(x_cat, tables):
    flat_tables = tables.reshape(_NF * _CARD, _DIM)
    offs = (jnp.arange(_NF, dtype=jnp.int32) * _CARD)[None, :]
    idx = (x_cat + offs).reshape(_NW, _RPW // _CHUNK, _CHUNK)
    mesh = plsc.VectorSubcoreMesh(core_axis_name="c", subcore_axis_name="s")
    out = pl.kernel(
        _gather_body,
        out_type=jax.ShapeDtypeStruct((_ROWS, _DIM), jnp.float32),
        mesh=mesh,
        scratch_types=[
            pltpu.VMEM((_RPW // _CHUNK, _CHUNK), jnp.int32),
            pltpu.VMEM((_K * _CHUNK, _DIM), jnp.float32),
            pltpu.SemaphoreType.DMA,
        ],
        compiler_params=pltpu.CompilerParams(use_tc_tiling_on_sc=False),
    )(idx, flat_tables)
    return out.reshape(_B, _NF * _DIM)


def kernel(x_cat, tables):
    return _cat_embed(x_cat, tables)
